# Initial kernel scaffold; baseline (speedup 1.0000x reference)
import jax
import jax.numpy as jnp
from jax.experimental import pallas as pl


def kernel(x, edge_index, edge_attr, batch, W, att_src, att_dst, bias, Wfc, bfc):
    raise NotImplementedError("write your pallas kernel here")


# speedup vs baseline: 9.8048x; 9.8048x over previous
import functools

import jax
import jax.numpy as jnp
from jax import lax
from jax.experimental import pallas as pl
from jax.experimental.pallas import tpu as pltpu
from jax.experimental.pallas import tpu_sc as plsc

N = 10000
NP = 10240
E = 320000
D = 128
DA = 144
NG = 16
NC = 2
NS = 16
TILES = NC * NS
EPT = E // TILES
GRP = 80
NCHUNK = EPT // GRP
STRIPE = NP // NS
RCH = 32


def _pre_body(x_ref, w_ref, asrc_ref, adst_ref, haug_ref, as_ref, ad_ref,
              selfw_ref):
    h = jnp.dot(x_ref[...], w_ref[...], preferred_element_type=jnp.float32)
    haug_ref[:, 0:D] = h
    lane = lax.broadcasted_iota(jnp.int32, (N, DA - D), 1)
    haug_ref[:, D:DA] = jnp.where(lane == 0, 1.0, 0.0).astype(jnp.float32)
    a_s = jnp.sum(h * asrc_ref[...], axis=1, keepdims=True)
    a_d = jnp.sum(h * adst_ref[...], axis=1, keepdims=True)
    as_ref[...] = a_s
    ad_ref[...] = a_d
    z = a_s + a_d
    z = jnp.maximum(z, 0.2 * z)
    selfw_ref[...] = 2.0 * jnp.exp(z)


_pre = pl.pallas_call(
    _pre_body,
    out_shape=(
        jax.ShapeDtypeStruct((N, DA), jnp.float32),
        jax.ShapeDtypeStruct((N, 1), jnp.float32),
        jax.ShapeDtypeStruct((N, 1), jnp.float32),
        jax.ShapeDtypeStruct((N, 1), jnp.float32),
    ),
)


def _sc_body(haug_hbm, as_hbm, ad_hbm, src2_hbm, dst2_hbm, zrow_hbm,
             parts_hbm, asv, adv, srcv, dstv, exv, augv, zbuf, acc, sem):
    c = lax.axis_index("c")
    s = lax.axis_index("s")
    tid = c * NS + s

    pltpu.sync_copy(as_hbm, asv)
    pltpu.sync_copy(ad_hbm, adv)
    pltpu.sync_copy(zrow_hbm, zbuf)

    @pl.loop(0, STRIPE // RCH)
    def _zero(i):
        pltpu.sync_copy(zbuf, acc.at[pl.ds(s * STRIPE + i * RCH, RCH)])

    plsc.subcore_barrier()

    @pl.loop(0, NCHUNK)
    def _chunk(k):
        rbase = tid * NCHUNK + k
        pltpu.sync_copy(src2_hbm.at[pl.ds(rbase, 1)], srcv)
        pltpu.sync_copy(dst2_hbm.at[pl.ds(rbase, 1)], dstv)

        cp = pltpu.async_copy(haug_hbm.at[srcv.at[0]], augv, sem)

        @pl.loop(0, GRP // 16)
        def _ex(g):
            s16 = srcv[0, pl.ds(g * 16, 16)]
            d16 = dstv[0, pl.ds(g * 16, 16)]
            z = (plsc.load_gather(asv, [s16])
                 + plsc.load_gather(adv, [d16]))
            z = jnp.maximum(z, 0.2 * z)
            exv[pl.ds(g * 16, 16)] = jnp.exp(z)

        cp.wait()

        @pl.loop(0, GRP // 16)
        def _scale(g):
            idx_e = g * 16 + lax.iota(jnp.int32, 16)
            ex16 = exv[pl.ds(g * 16, 16)]

            @pl.loop(0, DA, step=8)
            def _cols(d0):
                for dd in range(8):
                    dcol = jnp.full((16,), d0 + dd, jnp.int32)
                    v = plsc.load_gather(augv, [idx_e, dcol])
                    plsc.store_scatter(augv, [idx_e, dcol], v * ex16)

        pltpu.sync_copy(augv, acc.at[dstv.at[0]], add=True)

    plsc.subcore_barrier()

    @pl.loop(0, STRIPE // RCH)
    def _out(i):
        r = s * STRIPE + i * RCH
        pltpu.sync_copy(acc.at[pl.ds(r, RCH)], zbuf)
        pltpu.sync_copy(zbuf, parts_hbm.at[c, pl.ds(r, RCH)])


_sc_edge = pl.kernel(
    _sc_body,
    out_type=jax.ShapeDtypeStruct((NC, NP, DA), jnp.float32),
    mesh=plsc.VectorSubcoreMesh(core_axis_name="c", subcore_axis_name="s"),
    compiler_params=pltpu.CompilerParams(use_tc_tiling_on_sc=False,
                                         needs_layout_passes=False),
    scratch_types=[
        pltpu.VMEM((N,), jnp.float32),
        pltpu.VMEM((N,), jnp.float32),
        pltpu.VMEM((1, GRP), jnp.int32),
        pltpu.VMEM((1, GRP), jnp.int32),
        pltpu.VMEM((GRP,), jnp.float32),
        pltpu.VMEM((GRP, DA), jnp.float32),
        pltpu.VMEM((RCH, DA), jnp.float32),
        pltpu.VMEM_SHARED((NP, DA), jnp.float32),
        pltpu.SemaphoreType.DMA,
    ],
)


def _post_body(parts_ref, haug_ref, selfw_ref, batch_ref, bias_ref, wfc_ref,
               bfc_ref, res_ref):
    accs = parts_ref[0, 0:N, :] + parts_ref[1, 0:N, :]
    h = haug_ref[:, 0:D]
    selfw = selfw_ref[...]
    num = accs[:, 0:D] + selfw * h
    den = accs[:, D:D + 1] + selfw + 1e-16
    out = num / den + bias_ref[...]
    x2 = jnp.where(out > 0, out, jnp.exp(out) - 1.0)
    gid = lax.broadcasted_iota(jnp.int32, (NG, N), 0)
    onehot = jnp.where(gid == batch_ref[...], 1.0, 0.0).astype(jnp.float32)
    pooled_sum = jnp.dot(onehot, x2, preferred_element_type=jnp.float32)
    cnt = jnp.sum(onehot, axis=1, keepdims=True)
    pooled = pooled_sum / jnp.maximum(cnt, 1.0)
    res_ref[...] = (jnp.dot(pooled, wfc_ref[...],
                            preferred_element_type=jnp.float32)
                    + bfc_ref[...])


_post = pl.pallas_call(
    _post_body,
    out_shape=jax.ShapeDtypeStruct((NG, 1), jnp.float32),
)


def kernel(x, edge_index, edge_attr, batch, W, att_src, att_dst, bias, Wfc,
           bfc):
    del edge_attr
    src2 = edge_index[0].astype(jnp.int32).reshape(E // GRP, GRP)
    dst2 = edge_index[1].astype(jnp.int32).reshape(E // GRP, GRP)
    haug, a_s, a_d, selfw = _pre(x, W, att_src.reshape(1, D),
                                 att_dst.reshape(1, D))
    zrow = jnp.zeros((RCH, DA), jnp.float32)
    parts = _sc_edge(haug, a_s.reshape(-1), a_d.reshape(-1), src2, dst2,
                     zrow)
    res = _post(parts, haug, selfw, batch.astype(jnp.int32).reshape(1, N),
                bias.reshape(1, D), Wfc, bfc.reshape(1, 1))
    return res.reshape(-1)


# speedup vs baseline: 11.8850x; 1.2122x over previous
import functools

import jax
import jax.numpy as jnp
from jax import lax
from jax.experimental import pallas as pl
from jax.experimental.pallas import tpu as pltpu
from jax.experimental.pallas import tpu_sc as plsc

N = 10000
NP = 10240
E = 320000
D = 128
DA = 144
NG = 16
NC = 2
NS = 16
TILES = NC * NS
EPT = E // TILES
GRP = 80
NCHUNK = EPT // GRP
NSUP = 5
CPS = NCHUNK // NSUP
STRIPE = NP // NS
RCH = 32


def _pre_body(x_ref, w_ref, asrc_ref, adst_ref, haug_ref, ad16_ref,
              selfw_ref):
    h = jnp.dot(x_ref[...], w_ref[...], preferred_element_type=jnp.float32)
    haug_ref[:, 0:D] = h
    a_s = jnp.sum(h * asrc_ref[...], axis=1, keepdims=True)
    a_d = jnp.sum(h * adst_ref[...], axis=1, keepdims=True)
    lane = lax.broadcasted_iota(jnp.int32, (N, DA - D), 1)
    haug_ref[:, D:DA] = jnp.where(lane == 0, 1.0,
                                  jnp.where(lane == 1, a_s, 0.0))
    ad16_ref[...] = jnp.where(lane == 0, a_d, 0.0)
    z = a_s + a_d
    z = jnp.maximum(z, 0.2 * z)
    selfw_ref[...] = 2.0 * jnp.exp(z)


_pre = pl.pallas_call(
    _pre_body,
    out_shape=(
        jax.ShapeDtypeStruct((N, DA), jnp.float32),
        jax.ShapeDtypeStruct((N, 16), jnp.float32),
        jax.ShapeDtypeStruct((N, 1), jnp.float32),
    ),
)


def _sc_body(haug_hbm, ad16_hbm, sd_hbm, zrow_hbm, parts_hbm,
             sdv, sdv2, exv, augA, augB, adA, adB, zbuf, acc,
             semga, semgb, semsa, semsb):
    c = lax.axis_index("c")
    s = lax.axis_index("s")
    tid = c * NS + s
    colmask = jnp.where(lax.iota(jnp.int32, 16) == 0, 1.0, 0.0)

    pltpu.sync_copy(zrow_hbm, zbuf)

    @pl.loop(0, STRIPE // RCH)
    def _zero(i):
        pltpu.sync_copy(zbuf, acc.at[pl.ds(s * STRIPE + i * RCH, RCH)])

    plsc.subcore_barrier()

    def fire_g(k, aug, ad, sem):
        pltpu.async_copy(haug_hbm.at[sdv.at[k, 0]], aug, sem)
        pltpu.async_copy(ad16_hbm.at[sdv.at[k, 1]], ad, sem)

    def wait_g(aug, ad, sem):
        pltpu.make_async_copy(haug_hbm.at[pl.ds(0, GRP)], aug, sem).wait()
        pltpu.make_async_copy(ad16_hbm.at[pl.ds(0, GRP)], ad, sem).wait()

    def fire_s(k, aug, sem):
        pltpu.async_copy(aug, acc.at[sdv.at[k, 1]], sem, add=True)

    def wait_s(aug, sem):
        pltpu.make_async_copy(haug_hbm.at[pl.ds(0, GRP)], aug, sem).wait()

    def compute(aug, ad):
        @pl.loop(0, GRP // 16)
        def _ex(g):
            idx_e = g * 16 + lax.iota(jnp.int32, 16)
            z = (plsc.load_gather(aug, [idx_e, jnp.full((16,), D + 1,
                                                        jnp.int32)])
                 + plsc.load_gather(ad, [idx_e, jnp.zeros((16,),
                                                          jnp.int32)]))
            z = jnp.maximum(z, 0.2 * z)
            exv[pl.ds(g * 16, 16)] = jnp.exp(z)

        @pl.loop(0, GRP // 16)
        def _scale(g):
            idx_e = g * 16 + lax.iota(jnp.int32, 16)
            ex16 = exv[pl.ds(g * 16, 16)]

            @pl.loop(0, DA, step=8)
            def _cols(d0):
                for dd in range(8):
                    dcol = jnp.full((16,), d0 + dd, jnp.int32)
                    v = plsc.load_gather(aug, [idx_e, dcol])
                    plsc.store_scatter(aug, [idx_e, dcol], v * ex16)

    def process(k, aug, ad, semg, o_aug, o_ad, o_semg, o_sems, first):
        wait_g(aug, ad, semg)
        if first:
            @pl.when(k > 0)
            def _():
                wait_s(o_aug, o_sems)
        else:
            wait_s(o_aug, o_sems)
        fire_g(k + 1, o_aug, o_ad, o_semg)
        compute(aug, ad)

    @pl.loop(0, NSUP)
    def _super(q):
        pltpu.sync_copy(sd_hbm.at[tid, q], sdv)
        fire_g(0, augA, adA, semga)

        @pl.loop(0, (CPS - 1) // 2)
        def _pipe(i):
            k0 = 2 * i
            process(k0, augA, adA, semga, augB, adB, semgb, semsb, True)
            fire_s(k0, augA, semsa)
            process(k0 + 1, augB, adB, semgb, augA, adA, semga, semsa,
                    False)
            fire_s(k0 + 1, augB, semsb)

        wait_g(augA, adA, semga)
        compute(augA, adA)
        wait_s(augB, semsb)
        fire_s(CPS - 1, augA, semsa)
        wait_s(augA, semsa)

    plsc.subcore_barrier()

    @pl.loop(0, STRIPE // RCH)
    def _out(i):
        r = s * STRIPE + i * RCH
        pltpu.sync_copy(acc.at[pl.ds(r, RCH)], zbuf)
        pltpu.sync_copy(zbuf, parts_hbm.at[c, pl.ds(r, RCH)])


_sc_edge = pl.kernel(
    _sc_body,
    out_type=jax.ShapeDtypeStruct((NC, NP, DA), jnp.float32),
    mesh=plsc.VectorSubcoreMesh(core_axis_name="c", subcore_axis_name="s"),
    compiler_params=pltpu.CompilerParams(use_tc_tiling_on_sc=False,
                                         needs_layout_passes=False),
    scratch_types=[
        pltpu.VMEM((CPS, 2, GRP), jnp.int32),
        pltpu.VMEM((2, GRP), jnp.int32),
        pltpu.VMEM((GRP,), jnp.float32),
        pltpu.VMEM((GRP, DA), jnp.float32),
        pltpu.VMEM((GRP, DA), jnp.float32),
        pltpu.VMEM((GRP, 16), jnp.float32),
        pltpu.VMEM((GRP, 16), jnp.float32),
        pltpu.VMEM((RCH, DA), jnp.float32),
        pltpu.VMEM_SHARED((NP, DA), jnp.float32),
        pltpu.SemaphoreType.DMA,
        pltpu.SemaphoreType.DMA,
        pltpu.SemaphoreType.DMA,
        pltpu.SemaphoreType.DMA,
    ],
)


def _post_body(parts_ref, haug_ref, selfw_ref, batch_ref, bias_ref, wfc_ref,
               bfc_ref, res_ref):
    accs = parts_ref[0, 0:N, :] + parts_ref[1, 0:N, :]
    h = haug_ref[:, 0:D]
    selfw = selfw_ref[...]
    num = accs[:, 0:D] + selfw * h
    den = accs[:, D:D + 1] + selfw + 1e-16
    out = num / den + bias_ref[...]
    x2 = jnp.where(out > 0, out, jnp.exp(out) - 1.0)
    gid = lax.broadcasted_iota(jnp.int32, (NG, N), 0)
    onehot = jnp.where(gid == batch_ref[...], 1.0, 0.0).astype(jnp.float32)
    pooled_sum = jnp.dot(onehot, x2, preferred_element_type=jnp.float32)
    cnt = jnp.sum(onehot, axis=1, keepdims=True)
    pooled = pooled_sum / jnp.maximum(cnt, 1.0)
    res_ref[...] = (jnp.dot(pooled, wfc_ref[...],
                            preferred_element_type=jnp.float32)
                    + bfc_ref[...])


_post = pl.pallas_call(
    _post_body,
    out_shape=jax.ShapeDtypeStruct((NG, 1), jnp.float32),
)


def kernel(x, edge_index, edge_attr, batch, W, att_src, att_dst, bias, Wfc,
           bfc):
    del edge_attr
    src4 = edge_index[0].astype(jnp.int32).reshape(TILES, NSUP, CPS, GRP)
    dst4 = edge_index[1].astype(jnp.int32).reshape(TILES, NSUP, CPS, GRP)
    sd = jnp.stack([src4, dst4], axis=3)
    haug, ad16, selfw = _pre(x, W, att_src.reshape(1, D),
                             att_dst.reshape(1, D))
    zrow = jnp.zeros((RCH, DA), jnp.float32)
    parts = _sc_edge(haug, ad16, sd, zrow)
    res = _post(parts, haug, selfw, batch.astype(jnp.int32).reshape(1, N),
                bias.reshape(1, D), Wfc, bfc.reshape(1, 1))
    return res.reshape(-1)


# speedup vs baseline: 30.6211x; 2.5765x over previous
import functools

import jax
import jax.numpy as jnp
from jax import lax
from jax.experimental import pallas as pl
from jax.experimental.pallas import tpu as pltpu
from jax.experimental.pallas import tpu_sc as plsc

N = 10000
NP = 10240
E = 320000
D = 128
DA = 144
NG = 16
NC = 2
NS = 16
TILES = NC * NS
EPT = E // TILES
GRP = 80
NCHUNK = EPT // GRP
NSUP = 5
CPS = NCHUNK // NSUP
STRIPE = NP // NS
RCH = 32


def _pre_body(x_ref, w_ref, asrc_ref, adst_ref, haug_ref, ad16_ref,
              selfw_ref):
    h = jnp.dot(x_ref[...], w_ref[...], preferred_element_type=jnp.float32)
    haug_ref[:, 0:D] = h
    a_s = jnp.sum(h * asrc_ref[...], axis=1, keepdims=True)
    a_d = jnp.sum(h * adst_ref[...], axis=1, keepdims=True)
    lane = lax.broadcasted_iota(jnp.int32, (N, DA - D), 1)
    haug_ref[:, D:DA] = jnp.where(lane == 0, 1.0,
                                  jnp.where(lane == 1, a_s, 0.0))
    ad16_ref[...] = jnp.where(lane == 0, a_d, 0.0)
    z = a_s + a_d
    z = jnp.maximum(z, 0.2 * z)
    selfw_ref[...] = 2.0 * jnp.exp(z)


_pre = pl.pallas_call(
    _pre_body,
    out_shape=(
        jax.ShapeDtypeStruct((N, DA), jnp.float32),
        jax.ShapeDtypeStruct((N, 16), jnp.float32),
        jax.ShapeDtypeStruct((N, 1), jnp.float32),
    ),
)


def _sc_body(haug_hbm, ad16_hbm, sd_hbm, zrow_hbm, parts_hbm,
             sdv, sdv2, exv, augA, augB, adA, adB, zbuf, acc,
             semga, semgb, semsa, semsb):
    c = lax.axis_index("c")
    s = lax.axis_index("s")
    tid = c * NS + s
    colmask = jnp.where(lax.iota(jnp.int32, 16) == 0, 1.0, 0.0)

    pltpu.sync_copy(zrow_hbm, zbuf)

    @pl.loop(0, STRIPE // RCH)
    def _zero(i):
        pltpu.sync_copy(zbuf, acc.at[pl.ds(s * STRIPE + i * RCH, RCH)])

    plsc.subcore_barrier()

    def fire_g(k, aug, ad, sem):
        pltpu.async_copy(haug_hbm.at[sdv.at[k, 0]], aug, sem)
        pltpu.async_copy(ad16_hbm.at[sdv.at[k, 1]], ad, sem)

    def wait_g(aug, ad, sem):
        pltpu.make_async_copy(haug_hbm.at[pl.ds(0, GRP)], aug, sem).wait()
        pltpu.make_async_copy(ad16_hbm.at[pl.ds(0, GRP)], ad, sem).wait()

    def fire_s(k, aug, sem):
        pltpu.async_copy(aug, acc.at[sdv.at[k, 1]], sem, add=True)

    def wait_s(aug, sem):
        pltpu.make_async_copy(haug_hbm.at[pl.ds(0, GRP)], aug, sem).wait()

    def compute(aug, ad):
        @plsc.parallel_loop(0, GRP // 16, unroll=5)
        def _ex(g):
            idx_e = g * 16 + lax.iota(jnp.int32, 16)
            z = (plsc.load_gather(aug, [idx_e, jnp.full((16,), D + 1,
                                                        jnp.int32)])
                 + plsc.load_gather(ad, [idx_e, jnp.zeros((16,),
                                                          jnp.int32)]))
            z = jnp.maximum(z, 0.2 * z)
            exv[pl.ds(g * 16, 16)] = jnp.exp(z)

        @pl.loop(0, GRP // 16)
        def _scale(g):
            idx_e = g * 16 + lax.iota(jnp.int32, 16)
            ex16 = exv[pl.ds(g * 16, 16)]

            @plsc.parallel_loop(0, DA, unroll=8)
            def _cols(d):
                dcol = jnp.full((16,), d, jnp.int32)
                v = plsc.load_gather(aug, [idx_e, dcol])
                plsc.store_scatter(aug, [idx_e, dcol], v * ex16)

    def process(k, aug, ad, semg, o_aug, o_ad, o_semg, o_sems, first):
        wait_g(aug, ad, semg)
        if first:
            @pl.when(k > 0)
            def _():
                wait_s(o_aug, o_sems)
        else:
            wait_s(o_aug, o_sems)
        fire_g(k + 1, o_aug, o_ad, o_semg)
        compute(aug, ad)

    @pl.loop(0, NSUP)
    def _super(q):
        pltpu.sync_copy(sd_hbm.at[tid, q], sdv)
        fire_g(0, augA, adA, semga)

        @pl.loop(0, (CPS - 1) // 2)
        def _pipe(i):
            k0 = 2 * i
            process(k0, augA, adA, semga, augB, adB, semgb, semsb, True)
            fire_s(k0, augA, semsa)
            process(k0 + 1, augB, adB, semgb, augA, adA, semga, semsa,
                    False)
            fire_s(k0 + 1, augB, semsb)

        wait_g(augA, adA, semga)
        compute(augA, adA)
        wait_s(augB, semsb)
        fire_s(CPS - 1, augA, semsa)
        wait_s(augA, semsa)

    plsc.subcore_barrier()

    @pl.loop(0, STRIPE // RCH)
    def _out(i):
        r = s * STRIPE + i * RCH
        pltpu.sync_copy(acc.at[pl.ds(r, RCH)], zbuf)
        pltpu.sync_copy(zbuf, parts_hbm.at[c, pl.ds(r, RCH)])


_sc_edge = pl.kernel(
    _sc_body,
    out_type=jax.ShapeDtypeStruct((NC, NP, DA), jnp.float32),
    mesh=plsc.VectorSubcoreMesh(core_axis_name="c", subcore_axis_name="s"),
    compiler_params=pltpu.CompilerParams(use_tc_tiling_on_sc=False,
                                         needs_layout_passes=False),
    scratch_types=[
        pltpu.VMEM((CPS, 2, GRP), jnp.int32),
        pltpu.VMEM((2, GRP), jnp.int32),
        pltpu.VMEM((GRP,), jnp.float32),
        pltpu.VMEM((GRP, DA), jnp.float32),
        pltpu.VMEM((GRP, DA), jnp.float32),
        pltpu.VMEM((GRP, 16), jnp.float32),
        pltpu.VMEM((GRP, 16), jnp.float32),
        pltpu.VMEM((RCH, DA), jnp.float32),
        pltpu.VMEM_SHARED((NP, DA), jnp.float32),
        pltpu.SemaphoreType.DMA,
        pltpu.SemaphoreType.DMA,
        pltpu.SemaphoreType.DMA,
        pltpu.SemaphoreType.DMA,
    ],
)


def _post_body(parts_ref, haug_ref, selfw_ref, batch_ref, bias_ref, wfc_ref,
               bfc_ref, res_ref):
    accs = parts_ref[0, 0:N, :] + parts_ref[1, 0:N, :]
    h = haug_ref[:, 0:D]
    selfw = selfw_ref[...]
    num = accs[:, 0:D] + selfw * h
    den = accs[:, D:D + 1] + selfw + 1e-16
    out = num / den + bias_ref[...]
    x2 = jnp.where(out > 0, out, jnp.exp(out) - 1.0)
    gid = lax.broadcasted_iota(jnp.int32, (NG, N), 0)
    onehot = jnp.where(gid == batch_ref[...], 1.0, 0.0).astype(jnp.float32)
    pooled_sum = jnp.dot(onehot, x2, preferred_element_type=jnp.float32)
    cnt = jnp.sum(onehot, axis=1, keepdims=True)
    pooled = pooled_sum / jnp.maximum(cnt, 1.0)
    res_ref[...] = (jnp.dot(pooled, wfc_ref[...],
                            preferred_element_type=jnp.float32)
                    + bfc_ref[...])


_post = pl.pallas_call(
    _post_body,
    out_shape=jax.ShapeDtypeStruct((NG, 1), jnp.float32),
)


def kernel(x, edge_index, edge_attr, batch, W, att_src, att_dst, bias, Wfc,
           bfc):
    del edge_attr
    src4 = edge_index[0].astype(jnp.int32).reshape(TILES, NSUP, CPS, GRP)
    dst4 = edge_index[1].astype(jnp.int32).reshape(TILES, NSUP, CPS, GRP)
    sd = jnp.stack([src4, dst4], axis=3)
    haug, ad16, selfw = _pre(x, W, att_src.reshape(1, D),
                             att_dst.reshape(1, D))
    zrow = jnp.zeros((RCH, DA), jnp.float32)
    parts = _sc_edge(haug, ad16, sd, zrow)
    res = _post(parts, haug, selfw, batch.astype(jnp.int32).reshape(1, N),
                bias.reshape(1, D), Wfc, bfc.reshape(1, 1))
    return res.reshape(-1)


# speedup vs baseline: 32.8074x; 1.0714x over previous
import functools

import jax
import jax.numpy as jnp
from jax import lax
from jax.experimental import pallas as pl
from jax.experimental.pallas import tpu as pltpu
from jax.experimental.pallas import tpu_sc as plsc

N = 10000
NP = 10240
E = 320000
D = 128
DA = 144
NG = 16
NC = 2
NS = 16
TILES = NC * NS
EPT = E // TILES
GRP = 80
NCHUNK = EPT // GRP
NSUP = 5
CPS = NCHUNK // NSUP
STRIPE = NP // NS
RCH = 32


def _pre_body(x_ref, w_ref, asrc_ref, adst_ref, haug_ref, ad16_ref,
              selfw_ref):
    h = jnp.dot(x_ref[...], w_ref[...], preferred_element_type=jnp.float32)
    haug_ref[:, 0:D] = h
    a_s = jnp.sum(h * asrc_ref[...], axis=1, keepdims=True)
    a_d = jnp.sum(h * adst_ref[...], axis=1, keepdims=True)
    lane = lax.broadcasted_iota(jnp.int32, (N, DA - D), 1)
    haug_ref[:, D:DA] = jnp.where(lane == 0, 1.0,
                                  jnp.where(lane == 1, a_s, 0.0))
    ad16_ref[...] = jnp.where(lane == 0, a_d, 0.0)
    z = a_s + a_d
    z = jnp.maximum(z, 0.2 * z)
    selfw_ref[...] = 2.0 * jnp.exp(z)


_pre = pl.pallas_call(
    _pre_body,
    out_shape=(
        jax.ShapeDtypeStruct((N, DA), jnp.float32),
        jax.ShapeDtypeStruct((N, 16), jnp.float32),
        jax.ShapeDtypeStruct((N, 1), jnp.float32),
    ),
)


def _sc_body(haug_hbm, ad16_hbm, src2_hbm, dst2_hbm, zrow_hbm, parts_hbm,
             srcv, dstv, exv, augA, augB, adA, adB, acc,
             semga, semgb, semsa, semsb):
    c = lax.axis_index("c")
    s = lax.axis_index("s")
    tid = c * NS + s

    pltpu.sync_copy(zrow_hbm, acc.at[pl.ds(s * STRIPE, STRIPE)])

    plsc.subcore_barrier()

    def fire_g(k, aug, ad, sem):
        pltpu.async_copy(haug_hbm.at[srcv.at[k]], aug, sem)
        pltpu.async_copy(ad16_hbm.at[dstv.at[k]], ad, sem)

    def wait_g(aug, ad, sem):
        pltpu.make_async_copy(haug_hbm.at[pl.ds(0, GRP)], aug, sem).wait()
        pltpu.make_async_copy(ad16_hbm.at[pl.ds(0, GRP)], ad, sem).wait()

    def fire_s(k, aug, sem):
        pltpu.async_copy(aug, acc.at[dstv.at[k]], sem, add=True)

    def wait_s(aug, sem):
        pltpu.make_async_copy(haug_hbm.at[pl.ds(0, GRP)], aug, sem).wait()

    def compute(aug, ad):
        @plsc.parallel_loop(0, GRP // 16, unroll=5)
        def _ex(g):
            idx_e = g * 16 + lax.iota(jnp.int32, 16)
            z = (plsc.load_gather(aug, [idx_e, jnp.full((16,), D + 1,
                                                        jnp.int32)])
                 + plsc.load_gather(ad, [idx_e, jnp.zeros((16,),
                                                          jnp.int32)]))
            z = jnp.maximum(z, 0.2 * z)
            exv[pl.ds(g * 16, 16)] = jnp.exp(z)

        @plsc.parallel_loop(0, GRP // 16)
        def _scale(g):
            idx_e = g * 16 + lax.iota(jnp.int32, 16)
            ex16 = exv[pl.ds(g * 16, 16)]

            @plsc.parallel_loop(0, DA, unroll=8)
            def _cols(d):
                dcol = jnp.full((16,), d, jnp.int32)
                v = plsc.load_gather(aug, [idx_e, dcol])
                plsc.store_scatter(aug, [idx_e, dcol], v * ex16)

    def process(k, aug, ad, semg, o_aug, o_ad, o_semg, o_sems, first):
        wait_g(aug, ad, semg)
        if first:
            @pl.when(k > 0)
            def _():
                wait_s(o_aug, o_sems)
        else:
            wait_s(o_aug, o_sems)
        fire_g(k + 1, o_aug, o_ad, o_semg)
        compute(aug, ad)

    @pl.loop(0, NSUP)
    def _super(q):
        rbase = tid * NCHUNK + q * CPS
        pltpu.sync_copy(src2_hbm.at[pl.ds(rbase, CPS)], srcv)
        pltpu.sync_copy(dst2_hbm.at[pl.ds(rbase, CPS)], dstv)
        fire_g(0, augA, adA, semga)

        @pl.loop(0, (CPS - 1) // 2)
        def _pipe(i):
            k0 = 2 * i
            process(k0, augA, adA, semga, augB, adB, semgb, semsb, True)
            fire_s(k0, augA, semsa)
            process(k0 + 1, augB, adB, semgb, augA, adA, semga, semsa,
                    False)
            fire_s(k0 + 1, augB, semsb)

        wait_g(augA, adA, semga)
        compute(augA, adA)
        wait_s(augB, semsb)
        fire_s(CPS - 1, augA, semsa)
        wait_s(augA, semsa)

    plsc.subcore_barrier()

    pltpu.sync_copy(acc.at[pl.ds(s * STRIPE, STRIPE)],
                    parts_hbm.at[c, pl.ds(s * STRIPE, STRIPE)])


_sc_edge = pl.kernel(
    _sc_body,
    out_type=jax.ShapeDtypeStruct((NC, NP, DA), jnp.float32),
    mesh=plsc.VectorSubcoreMesh(core_axis_name="c", subcore_axis_name="s"),
    compiler_params=pltpu.CompilerParams(use_tc_tiling_on_sc=False,
                                         needs_layout_passes=False),
    scratch_types=[
        pltpu.VMEM((CPS, GRP), jnp.int32),
        pltpu.VMEM((CPS, GRP), jnp.int32),
        pltpu.VMEM((GRP,), jnp.float32),
        pltpu.VMEM((GRP, DA), jnp.float32),
        pltpu.VMEM((GRP, DA), jnp.float32),
        pltpu.VMEM((GRP, 16), jnp.float32),
        pltpu.VMEM((GRP, 16), jnp.float32),
        pltpu.VMEM_SHARED((NP, DA), jnp.float32),
        pltpu.SemaphoreType.DMA,
        pltpu.SemaphoreType.DMA,
        pltpu.SemaphoreType.DMA,
        pltpu.SemaphoreType.DMA,
    ],
)


def _post_body(parts_ref, haug_ref, selfw_ref, batch_ref, bias_ref, wfc_ref,
               bfc_ref, res_ref):
    accs = parts_ref[0, 0:N, :] + parts_ref[1, 0:N, :]
    h = haug_ref[:, 0:D]
    selfw = selfw_ref[...]
    num = accs[:, 0:D] + selfw * h
    den = accs[:, D:D + 1] + selfw + 1e-16
    out = num / den + bias_ref[...]
    x2 = jnp.where(out > 0, out, jnp.exp(out) - 1.0)
    gid = lax.broadcasted_iota(jnp.int32, (NG, N), 0)
    onehot = jnp.where(gid == batch_ref[...], 1.0, 0.0).astype(jnp.float32)
    pooled_sum = jnp.dot(onehot, x2, preferred_element_type=jnp.float32)
    cnt = jnp.sum(onehot, axis=1, keepdims=True)
    pooled = pooled_sum / jnp.maximum(cnt, 1.0)
    res_ref[...] = (jnp.dot(pooled, wfc_ref[...],
                            preferred_element_type=jnp.float32)
                    + bfc_ref[...])


_post = pl.pallas_call(
    _post_body,
    out_shape=jax.ShapeDtypeStruct((NG, 1), jnp.float32),
)


def kernel(x, edge_index, edge_attr, batch, W, att_src, att_dst, bias, Wfc,
           bfc):
    del edge_attr
    src2 = edge_index[0].astype(jnp.int32).reshape(E // GRP, GRP)
    dst2 = edge_index[1].astype(jnp.int32).reshape(E // GRP, GRP)
    haug, ad16, selfw = _pre(x, W, att_src.reshape(1, D),
                             att_dst.reshape(1, D))
    zrow = jnp.zeros((STRIPE, DA), jnp.float32)
    parts = _sc_edge(haug, ad16, src2, dst2, zrow)
    res = _post(parts, haug, selfw, batch.astype(jnp.int32).reshape(1, N),
                bias.reshape(1, D), Wfc, bfc.reshape(1, 1))
    return res.reshape(-1)
